# baseline (speedup 1.0000x reference)
import functools

import jax
import jax.numpy as jnp
from jax import lax
from jax.experimental import pallas as pl
from jax.experimental.pallas import tpu as pltpu
from jax.experimental.pallas import tpu_sc as plsc

HEADS = 12
NWIN = 16
TOPK = 4
TAU = 2.0
VTH = 1.0
EPS = 1e-5

_INTERPRET = False


def _stats_route_body(g_ref, be_ref, x_ref, wqk_ref, bqk_ref,
                      o_ref, a_ref, ms_scr, x_scr, *, n_blocks, n_rows,
                      w, d, bn, r):
    i = pl.program_id(0)
    rows_per_blk = n_rows // n_blocks
    regs_per_blk = rows_per_blk // w

    @pl.when(i < n_blocks)
    def _stats():
        xb = x_ref[...]
        part = jnp.stack(
            [jnp.sum(xb, axis=0), jnp.sum(xb * xb, axis=0)], axis=0)
        o_ref[pl.ds(i, 1)] = part[None]
        x_scr[pl.ds(i * rows_per_blk, rows_per_blk), :] = xb

    @pl.when(i >= n_blocks)
    def _ms():
        blk = i - n_blocks
        ssum = jnp.sum(o_ref[...], axis=0)
        mean = ssum[0] * (1.0 / n_rows)
        var = ssum[1] * (1.0 / n_rows) - mean * mean
        scl = g_ref[0] * jax.lax.rsqrt(var + EPS)
        shf = be_ref[0] - mean * scl
        xb = x_scr[pl.ds(blk * rows_per_blk, rows_per_blk), :]
        spk = _spike(xb * scl[None, :] + shf[None, :])
        for q in range(regs_per_blk):
            ms_scr[pl.ds(blk * regs_per_blk + q, 1), :] = (
                jnp.sum(spk[q * w:(q + 1) * w], axis=0)[None, :] * (1.0 / w))

        @pl.when(blk == n_blocks - 1)
        def _affinity():
            ms = ms_scr[...]
            qkr = jax.lax.dot_general(
                ms, wqk_ref[...], (((1,), (0,)), ((), ())),
                preferred_element_type=jnp.float32) + bqk_ref[...]
            qr = qkr[:, :d]
            kr = qkr[:, d:]
            for bb in range(bn):
                a_ref[bb] = jax.lax.dot_general(
                    qr[bb * r:(bb + 1) * r], kr[bb * r:(bb + 1) * r],
                    (((1,), (1,)), ((), ())),
                    preferred_element_type=jnp.float32)


_GATHER_DNUMS = lax.GatherDimensionNumbers(
    offset_dims=(), collapsed_slice_dims=(0,), start_index_map=(0,))


def _vgather(v, perm):
    return lax.gather(v, perm[:, None], _GATHER_DNUMS, (1,),
                      mode=lax.GatherScatterMode.PROMISE_IN_BOUNDS)


def _sc_topk(a_hbm, out_hbm, a_v, idx_v):
    nc = 2
    wid = lax.axis_index("s") * nc + lax.axis_index("c")
    ids = lax.iota(jnp.int32, 16)
    for t in range(2):
        row = wid * 2 + t
        pltpu.sync_copy(a_hbm.at[row], a_v)
        keys = a_v[...]
        out = ids * 0
        for p in range(TOPK):
            mx = keys
            for sft in (8, 4, 2, 1):
                mx = jnp.maximum(mx, _vgather(mx, jnp.bitwise_xor(ids, sft)))
            cand = jnp.where(keys >= mx, ids, 16)
            for sft in (8, 4, 2, 1):
                cand = jnp.minimum(
                    cand, _vgather(cand, jnp.bitwise_xor(ids, sft)))
            out = out + jnp.where(ids == p, cand, 0)
            keys = jnp.where(ids == cand, jnp.float32(-3.0e38), keys)
        idx_v[...] = out
        pltpu.sync_copy(idx_v, out_hbm.at[row])


def _bn_coeffs(stats_ref, g_ref, be_ref, n_rows):
    s = jnp.sum(stats_ref[...], axis=0)
    mean = s[0] * (1.0 / n_rows)
    var = s[1] * (1.0 / n_rows) - mean * mean
    scl = g_ref[0] * jax.lax.rsqrt(var + EPS)
    shf = be_ref[0] - mean * scl
    return scl, shf


def _spike(xn):
    v = xn / TAU
    return (v - VTH >= 0.0).astype(jnp.float32)


def _mega_body(idx_ref, stats_ref, g_ref, be_ref, xb_ref, wb_ref,
               bq_ref, wo_ref, bo_ref, sc_ref,
               y_ref, st_ref, q_scr, kv_scr,
               *, n_rows, w, d, r, dh, rpb):
    i = pl.program_id(0)
    nq = r // rpb
    c = jax.lax.rem(i, 2 * nq)
    bidx = i // (2 * nq)

    @pl.when(c < nq)
    def _qkv():
        scl, shf = _bn_coeffs(stats_ref, g_ref, be_ref, n_rows)
        xb = xb_ref[0].reshape(rpb * w, d)
        spk = _spike(xb * scl[None, :] + shf[None, :])
        qkv = jax.lax.dot_general(
            spk.astype(jnp.bfloat16), wb_ref[...],
            (((1,), (0,)), ((), ())), preferred_element_type=jnp.float32)
        qkv = qkv + bq_ref[...]
        q_scr[pl.ds(rpb * c, rpb)] = (
            qkv[:, :d].astype(jnp.bfloat16).reshape(rpb, w, d))
        kv_scr[pl.ds(rpb * c, rpb)] = (
            qkv[:, d:].astype(jnp.bfloat16).reshape(rpb, w, 2 * d))

    @pl.when(c >= nq)
    def _attn():
        ja = c - nq
        qs = []
        kcs = []
        vcs = []
        for u in range(rpb):
            reg = rpb * ja + u
            qs.append(q_scr[reg] * jnp.bfloat16(dh ** -0.5))
            kvc = jnp.concatenate(
                [kv_scr[idx_ref[bidx, reg, t]] for t in range(TOPK)],
                axis=0)
            kcs.append(kvc[:, :d])
            vcs.append(kvc[:, d:])
        nk = kcs[0].shape[0]
        ones_m = jnp.ones((nk, 8), jnp.bfloat16)
        heads = [[] for _ in range(rpb)]
        for h in range(HEADS):
            sl = slice(h * dh, (h + 1) * dh)
            for u in range(rpb):
                s = jax.lax.dot_general(
                    qs[u][:, sl], kcs[u][:, sl], (((1,), (1,)), ((), ())),
                    preferred_element_type=jnp.float32).astype(jnp.bfloat16)
                m = jnp.max(s, axis=1, keepdims=True)
                p = jnp.exp(s - m)
                l = jax.lax.dot_general(
                    p, ones_m, (((1,), (0,)), ((), ())),
                    preferred_element_type=jnp.float32)[:, :1]
                o_h = jax.lax.dot_general(
                    p, vcs[u][:, sl], (((1,), (0,)), ((), ())),
                    preferred_element_type=jnp.float32)
                heads[u].append((o_h * (1.0 / l)).astype(jnp.bfloat16))
        ssum = None
        ssq = None
        for u in range(rpb):
            oc = jnp.concatenate(heads[u], axis=1)
            res = jax.lax.dot_general(
                oc, wo_ref[...], (((1,), (0,)), ((), ())),
                preferred_element_type=jnp.float32) + bo_ref[...]
            y = xb_ref[0, u] + sc_ref[0, 0] * res
            y_ref[0, u] = y
            us = jnp.sum(y, axis=0)
            uq = jnp.sum(y * y, axis=0)
            ssum = us if ssum is None else ssum + us
            ssq = uq if ssq is None else ssq + uq
        st_ref[0, 0, :] = ssum
        st_ref[0, 1, :] = ssq


def _ffn_body(stats_ref, g_ref, be_ref, y_ref, w1_ref, b1_ref,
              w2_ref, b2_ref, sc_ref, o_ref, *, n_rows):
    scl, shf = _bn_coeffs(stats_ref, g_ref, be_ref, n_rows)
    yb = y_ref[...]
    spk = _spike(yb * scl[None, :] + shf[None, :])
    h = jax.lax.dot_general(
        spk.astype(jnp.bfloat16), w1_ref[...], (((1,), (0,)), ((), ())),
        preferred_element_type=jnp.float32) + b1_ref[...]
    g = 0.5 * h * (1.0 + jax.lax.erf(h * (2.0 ** -0.5)))
    f = jax.lax.dot_general(
        g.astype(jnp.bfloat16), w2_ref[...], (((1,), (0,)), ((), ())),
        preferred_element_type=jnp.float32) + b2_ref[...]
    o_ref[...] = yb + sc_ref[0, 0] * f


def kernel(x, Lt, b, L, dim, bn1_gamma, bn1_beta, W_qkv, b_qkv, W_o, b_o,
           bn2_gamma, bn2_beta, W1, b1, W2, b2, scale):
    Lt_s, b_s, L_s, d = x.shape
    bn = Lt_s * b_s
    r = NWIN
    w = L_s // r
    n = bn * L_s
    dh = d // HEADS
    dff = W1.shape[1]
    rpb = 8
    cyc = 2 * (r // rpb)

    x2d = x.reshape(n, d)
    x4 = x.reshape(bn, r, w, d)
    n_blocks = 8

    g1 = bn1_gamma.reshape(1, d)
    be1 = bn1_beta.reshape(1, d)
    wqk = W_qkv[:, :2 * d]
    bqk = b_qkv[:2 * d].reshape(1, 2 * d)

    stats1, a_mat = pl.pallas_call(
        functools.partial(_stats_route_body, n_blocks=n_blocks, n_rows=n,
                          w=w, d=d, bn=bn, r=r),
        grid=(2 * n_blocks,),
        in_specs=[
            pl.BlockSpec((1, d), lambda i: (0, 0)),
            pl.BlockSpec((1, d), lambda i: (0, 0)),
            pl.BlockSpec((n // n_blocks, d),
                         lambda i: (jnp.minimum(i, n_blocks - 1), 0)),
            pl.BlockSpec((d, 2 * d), lambda i: (0, 0)),
            pl.BlockSpec((1, 2 * d), lambda i: (0, 0)),
        ],
        out_specs=[
            pl.BlockSpec((n_blocks, 2, d), lambda i: (0, 0, 0)),
            pl.BlockSpec((bn, r, r), lambda i: (0, 0, 0)),
        ],
        out_shape=[
            jax.ShapeDtypeStruct((n_blocks, 2, d), jnp.float32),
            jax.ShapeDtypeStruct((bn, r, r), jnp.float32),
        ],
        scratch_shapes=[pltpu.VMEM((bn * r, d), jnp.float32),
                        pltpu.VMEM((n, d), jnp.float32)],
        interpret=_INTERPRET,
    )(g1, be1, x2d, wqk, bqk)

    idxfull = pl.kernel(
        _sc_topk,
        out_type=jax.ShapeDtypeStruct((bn * r, 16), jnp.int32),
        mesh=plsc.VectorSubcoreMesh(core_axis_name="c",
                                    subcore_axis_name="s"),
        scratch_types=[pltpu.VMEM((16,), jnp.float32),
                       pltpu.VMEM((16,), jnp.int32)],
    )(a_mat.reshape(bn * r, r))
    idx = idxfull.reshape(bn, r, 16)[:, :, :TOPK]

    wqkv_bf = W_qkv.astype(jnp.bfloat16)
    bq2 = b_qkv.reshape(1, 3 * d)
    wo_bf = W_o.astype(jnp.bfloat16)
    bo2 = b_o.reshape(1, d)
    sc2 = scale.reshape(1, 1)

    nq = r // rpb

    def _b(i):
        return i // cyc

    def _c(i):
        return jax.lax.rem(i, cyc)

    def x_map(i, s):
        return (_b(i), jax.lax.rem(_c(i), nq), 0, 0)

    def att_map(i, s):
        return (_b(i), jnp.clip(_c(i) - nq, 0, nq - 1), 0, 0)

    def st_map(i, s):
        return (_b(i) * nq + jnp.clip(_c(i) - nq, 0, nq - 1), 0, 0)

    const2 = lambda i, s: (0, 0)
    const3 = lambda i, s: (0, 0, 0)
    y4, stats2 = pl.pallas_call(
        functools.partial(_mega_body, n_rows=n, w=w, d=d, r=r, dh=dh,
                          rpb=rpb),
        grid_spec=pltpu.PrefetchScalarGridSpec(
            num_scalar_prefetch=1,
            grid=(bn * cyc,),
            in_specs=[
                pl.BlockSpec((n_blocks, 2, d), const3),
                pl.BlockSpec((1, d), const2),
                pl.BlockSpec((1, d), const2),
                pl.BlockSpec((1, rpb, w, d), x_map),
                pl.BlockSpec((d, 3 * d), const2),
                pl.BlockSpec((1, 3 * d), const2),
                pl.BlockSpec((d, d), const2),
                pl.BlockSpec((1, d), const2),
                pl.BlockSpec((1, 1), const2),
            ],
            out_specs=[
                pl.BlockSpec((1, rpb, w, d), att_map),
                pl.BlockSpec((1, 2, d), st_map),
            ],
            scratch_shapes=[
                pltpu.VMEM((r, w, d), jnp.bfloat16),
                pltpu.VMEM((r, w, 2 * d), jnp.bfloat16),
            ],
        ),
        out_shape=[
            jax.ShapeDtypeStruct((bn, r, w, d), jnp.float32),
            jax.ShapeDtypeStruct((bn * r // rpb, 2, d), jnp.float32),
        ],
        interpret=_INTERPRET,
    )(idx, stats1, g1, be1, x4, wqkv_bf, bq2, wo_bf, bo2, sc2)

    y2d = y4.reshape(n, d)

    w1_bf = W1.astype(jnp.bfloat16)
    w2_bf = W2.astype(jnp.bfloat16)
    g2 = bn2_gamma.reshape(1, d)
    be2 = bn2_beta.reshape(1, d)
    b12 = b1.reshape(1, dff)
    b22 = b2.reshape(1, d)
    n_blk = 16
    blk = n // n_blk
    out2d = pl.pallas_call(
        functools.partial(_ffn_body, n_rows=n),
        grid=(n_blk,),
        in_specs=[
            pl.BlockSpec((bn * r // rpb, 2, d), lambda i: (0, 0, 0)),
            pl.BlockSpec((1, d), lambda i: (0, 0)),
            pl.BlockSpec((1, d), lambda i: (0, 0)),
            pl.BlockSpec((blk, d), lambda i: (i, 0)),
            pl.BlockSpec((d, dff), lambda i: (0, 0)),
            pl.BlockSpec((1, dff), lambda i: (0, 0)),
            pl.BlockSpec((dff, d), lambda i: (0, 0)),
            pl.BlockSpec((1, d), lambda i: (0, 0)),
            pl.BlockSpec((1, 1), lambda i: (0, 0)),
        ],
        out_specs=pl.BlockSpec((blk, d), lambda i: (i, 0)),
        out_shape=jax.ShapeDtypeStruct((n, d), jnp.float32),
        interpret=_INTERPRET,
    )(stats2, g2, be2, y2d, w1_bf, b12, w2_bf, b22, sc2)

    return out2d.reshape(Lt_s, b_s, L_s, d)


# speedup vs baseline: 1.0038x; 1.0038x over previous
import functools

import jax
import jax.numpy as jnp
from jax import lax
from jax.experimental import pallas as pl
from jax.experimental.pallas import tpu as pltpu
from jax.experimental.pallas import tpu_sc as plsc

HEADS = 12
NWIN = 16
TOPK = 4
TAU = 2.0
VTH = 1.0
EPS = 1e-5

def _stats_route_body(g_ref, be_ref, x_ref, wqk_ref, bqk_ref,
                      o_ref, a_ref, ms_scr, x_scr, *, n_blocks, n_rows,
                      w, d, bn, r):
    i = pl.program_id(0)
    rows_per_blk = n_rows // n_blocks
    regs_per_blk = rows_per_blk // w

    @pl.when(i < n_blocks)
    def _stats():
        xb = x_ref[...]
        part = jnp.stack(
            [jnp.sum(xb, axis=0), jnp.sum(xb * xb, axis=0)], axis=0)
        o_ref[pl.ds(i, 1)] = part[None]
        x_scr[pl.ds(i * rows_per_blk, rows_per_blk), :] = xb

    @pl.when(i >= n_blocks)
    def _ms():
        blk = i - n_blocks
        ssum = jnp.sum(o_ref[...], axis=0)
        mean = ssum[0] * (1.0 / n_rows)
        var = ssum[1] * (1.0 / n_rows) - mean * mean
        scl = g_ref[0] * jax.lax.rsqrt(var + EPS)
        shf = be_ref[0] - mean * scl
        xb = x_scr[pl.ds(blk * rows_per_blk, rows_per_blk), :]
        spk = _spike(xb * scl[None, :] + shf[None, :])
        for q in range(regs_per_blk):
            ms_scr[pl.ds(blk * regs_per_blk + q, 1), :] = (
                jnp.sum(spk[q * w:(q + 1) * w], axis=0)[None, :] * (1.0 / w))

        @pl.when(blk == n_blocks - 1)
        def _affinity():
            ms = ms_scr[...]
            qkr = jax.lax.dot_general(
                ms, wqk_ref[...], (((1,), (0,)), ((), ())),
                preferred_element_type=jnp.float32) + bqk_ref[...]
            qr = qkr[:, :d]
            kr = qkr[:, d:]
            for bb in range(bn):
                a_ref[bb] = jax.lax.dot_general(
                    qr[bb * r:(bb + 1) * r], kr[bb * r:(bb + 1) * r],
                    (((1,), (1,)), ((), ())),
                    preferred_element_type=jnp.float32)


_GATHER_DNUMS = lax.GatherDimensionNumbers(
    offset_dims=(), collapsed_slice_dims=(0,), start_index_map=(0,))


def _vgather(v, perm):
    return lax.gather(v, perm[:, None], _GATHER_DNUMS, (1,),
                      mode=lax.GatherScatterMode.PROMISE_IN_BOUNDS)


def _sc_topk(a_hbm, out_hbm, a_v, idx_v):
    nc = 2
    wid = lax.axis_index("s") * nc + lax.axis_index("c")
    ids = lax.iota(jnp.int32, 16)
    for t in range(2):
        row = wid * 2 + t
        pltpu.sync_copy(a_hbm.at[row], a_v)
        keys = a_v[...]
        out = ids * 0
        for p in range(TOPK):
            mx = keys
            for sft in (8, 4, 2, 1):
                mx = jnp.maximum(mx, _vgather(mx, jnp.bitwise_xor(ids, sft)))
            cand = jnp.where(keys >= mx, ids, 16)
            for sft in (8, 4, 2, 1):
                cand = jnp.minimum(
                    cand, _vgather(cand, jnp.bitwise_xor(ids, sft)))
            out = out + jnp.where(ids == p, cand, 0)
            keys = jnp.where(ids == cand, jnp.float32(-3.0e38), keys)
        idx_v[...] = out
        pltpu.sync_copy(idx_v, out_hbm.at[row])


def _bn_coeffs(stats_ref, g_ref, be_ref, n_rows):
    s = jnp.sum(stats_ref[...], axis=0)
    mean = s[0] * (1.0 / n_rows)
    var = s[1] * (1.0 / n_rows) - mean * mean
    scl = g_ref[0] * jax.lax.rsqrt(var + EPS)
    shf = be_ref[0] - mean * scl
    return scl, shf


def _spike(xn):
    v = xn / TAU
    return (v - VTH >= 0.0).astype(jnp.float32)


def _mega_body(idx_ref, stats_ref, g_ref, be_ref, xb_ref, wb_ref,
               bq_ref, wo_ref, bo_ref, sc_ref,
               y_ref, st_ref, q_scr, kv_scr,
               *, n_rows, w, d, r, dh, rpb):
    i = pl.program_id(0)
    nq = r // rpb
    c = jax.lax.rem(i, 2 * nq)
    bidx = i // (2 * nq)

    @pl.when(c < nq)
    def _qkv():
        scl, shf = _bn_coeffs(stats_ref, g_ref, be_ref, n_rows)
        xb = xb_ref[0].reshape(rpb * w, d)
        spk = _spike(xb * scl[None, :] + shf[None, :])
        qkv = jax.lax.dot_general(
            spk.astype(jnp.bfloat16), wb_ref[...],
            (((1,), (0,)), ((), ())), preferred_element_type=jnp.float32)
        qkv = qkv + bq_ref[...]
        q_scr[pl.ds(rpb * c, rpb)] = (
            qkv[:, :d].astype(jnp.bfloat16).reshape(rpb, w, d))
        kv_scr[pl.ds(rpb * c, rpb)] = (
            qkv[:, d:].astype(jnp.bfloat16).reshape(rpb, w, 2 * d))

    @pl.when(c >= nq)
    def _attn():
        ja = c - nq
        qs = []
        kcs = []
        vcs = []
        for u in range(rpb):
            reg = rpb * ja + u
            qs.append(q_scr[reg] * jnp.bfloat16(dh ** -0.5))
            kvc = jnp.concatenate(
                [kv_scr[idx_ref[bidx, reg, t]] for t in range(TOPK)],
                axis=0)
            kcs.append(kvc[:, :d])
            vcs.append(kvc[:, d:])
        nk = kcs[0].shape[0]
        ones_m = jnp.ones((nk, 8), jnp.bfloat16)
        heads = [[] for _ in range(rpb)]
        for h in range(HEADS):
            sl = slice(h * dh, (h + 1) * dh)
            for u in range(rpb):
                s = jax.lax.dot_general(
                    qs[u][:, sl], kcs[u][:, sl], (((1,), (1,)), ((), ())),
                    preferred_element_type=jnp.float32).astype(jnp.bfloat16)
                m = jnp.max(s, axis=1, keepdims=True)
                p = jnp.exp(s - m)
                l = jax.lax.dot_general(
                    p, ones_m, (((1,), (0,)), ((), ())),
                    preferred_element_type=jnp.float32)[:, :1]
                o_h = jax.lax.dot_general(
                    p, vcs[u][:, sl], (((1,), (0,)), ((), ())),
                    preferred_element_type=jnp.float32)
                heads[u].append((o_h * (1.0 / l)).astype(jnp.bfloat16))
        ssum = None
        ssq = None
        for u in range(rpb):
            oc = jnp.concatenate(heads[u], axis=1)
            res = jax.lax.dot_general(
                oc, wo_ref[...], (((1,), (0,)), ((), ())),
                preferred_element_type=jnp.float32) + bo_ref[...]
            y = xb_ref[0, u] + sc_ref[0, 0] * res
            y_ref[0, u] = y
            us = jnp.sum(y, axis=0)
            uq = jnp.sum(y * y, axis=0)
            ssum = us if ssum is None else ssum + us
            ssq = uq if ssq is None else ssq + uq
        st_ref[0, 0, :] = ssum
        st_ref[0, 1, :] = ssq


def _ffn_body(stats_ref, g_ref, be_ref, y_ref, w1_ref, b1_ref,
              w2_ref, b2_ref, sc_ref, o_ref, *, n_rows):
    scl, shf = _bn_coeffs(stats_ref, g_ref, be_ref, n_rows)
    yb = y_ref[...]
    spk = _spike(yb * scl[None, :] + shf[None, :])
    h = jax.lax.dot_general(
        spk.astype(jnp.bfloat16), w1_ref[...], (((1,), (0,)), ((), ())),
        preferred_element_type=jnp.float32) + b1_ref[...]
    g = 0.5 * h * (1.0 + jax.lax.erf(h * (2.0 ** -0.5)))
    f = jax.lax.dot_general(
        g.astype(jnp.bfloat16), w2_ref[...], (((1,), (0,)), ((), ())),
        preferred_element_type=jnp.float32) + b2_ref[...]
    o_ref[...] = yb + sc_ref[0, 0] * f


def kernel(x, Lt, b, L, dim, bn1_gamma, bn1_beta, W_qkv, b_qkv, W_o, b_o,
           bn2_gamma, bn2_beta, W1, b1, W2, b2, scale):
    Lt_s, b_s, L_s, d = x.shape
    bn = Lt_s * b_s
    r = NWIN
    w = L_s // r
    n = bn * L_s
    dh = d // HEADS
    dff = W1.shape[1]
    rpb = 4
    cyc = 2 * (r // rpb)

    x2d = x.reshape(n, d)
    x4 = x.reshape(bn, r, w, d)
    n_blocks = 8

    g1 = bn1_gamma.reshape(1, d)
    be1 = bn1_beta.reshape(1, d)
    wqk = W_qkv[:, :2 * d]
    bqk = b_qkv[:2 * d].reshape(1, 2 * d)

    stats1, a_mat = pl.pallas_call(
        functools.partial(_stats_route_body, n_blocks=n_blocks, n_rows=n,
                          w=w, d=d, bn=bn, r=r),
        grid=(2 * n_blocks,),
        in_specs=[
            pl.BlockSpec((1, d), lambda i: (0, 0)),
            pl.BlockSpec((1, d), lambda i: (0, 0)),
            pl.BlockSpec((n // n_blocks, d),
                         lambda i: (jnp.minimum(i, n_blocks - 1), 0)),
            pl.BlockSpec((d, 2 * d), lambda i: (0, 0)),
            pl.BlockSpec((1, 2 * d), lambda i: (0, 0)),
        ],
        out_specs=[
            pl.BlockSpec((n_blocks, 2, d), lambda i: (0, 0, 0)),
            pl.BlockSpec((bn, r, r), lambda i: (0, 0, 0)),
        ],
        out_shape=[
            jax.ShapeDtypeStruct((n_blocks, 2, d), jnp.float32),
            jax.ShapeDtypeStruct((bn, r, r), jnp.float32),
        ],
        scratch_shapes=[pltpu.VMEM((bn * r, d), jnp.float32),
                        pltpu.VMEM((n, d), jnp.float32)],
    )(g1, be1, x2d, wqk, bqk)

    idxfull = pl.kernel(
        _sc_topk,
        out_type=jax.ShapeDtypeStruct((bn * r, 16), jnp.int32),
        mesh=plsc.VectorSubcoreMesh(core_axis_name="c",
                                    subcore_axis_name="s"),
        scratch_types=[pltpu.VMEM((16,), jnp.float32),
                       pltpu.VMEM((16,), jnp.int32)],
    )(a_mat.reshape(bn * r, r))
    idx = idxfull.reshape(bn, r, 16)[:, :, :TOPK]

    wqkv_bf = W_qkv.astype(jnp.bfloat16)
    bq2 = b_qkv.reshape(1, 3 * d)
    wo_bf = W_o.astype(jnp.bfloat16)
    bo2 = b_o.reshape(1, d)
    sc2 = scale.reshape(1, 1)

    nq = r // rpb

    def _b(i):
        return i // cyc

    def _c(i):
        return jax.lax.rem(i, cyc)

    def x_map(i, s):
        return (_b(i), jax.lax.rem(_c(i), nq), 0, 0)

    def att_map(i, s):
        return (_b(i), jnp.clip(_c(i) - nq, 0, nq - 1), 0, 0)

    def st_map(i, s):
        return (_b(i) * nq + jnp.clip(_c(i) - nq, 0, nq - 1), 0, 0)

    const2 = lambda i, s: (0, 0)
    const3 = lambda i, s: (0, 0, 0)
    y4, stats2 = pl.pallas_call(
        functools.partial(_mega_body, n_rows=n, w=w, d=d, r=r, dh=dh,
                          rpb=rpb),
        grid_spec=pltpu.PrefetchScalarGridSpec(
            num_scalar_prefetch=1,
            grid=(bn * cyc,),
            in_specs=[
                pl.BlockSpec((n_blocks, 2, d), const3),
                pl.BlockSpec((1, d), const2),
                pl.BlockSpec((1, d), const2),
                pl.BlockSpec((1, rpb, w, d), x_map),
                pl.BlockSpec((d, 3 * d), const2),
                pl.BlockSpec((1, 3 * d), const2),
                pl.BlockSpec((d, d), const2),
                pl.BlockSpec((1, d), const2),
                pl.BlockSpec((1, 1), const2),
            ],
            out_specs=[
                pl.BlockSpec((1, rpb, w, d), att_map),
                pl.BlockSpec((1, 2, d), st_map),
            ],
            scratch_shapes=[
                pltpu.VMEM((r, w, d), jnp.bfloat16),
                pltpu.VMEM((r, w, 2 * d), jnp.bfloat16),
            ],
        ),
        out_shape=[
            jax.ShapeDtypeStruct((bn, r, w, d), jnp.float32),
            jax.ShapeDtypeStruct((bn * r // rpb, 2, d), jnp.float32),
        ],
    )(idx, stats1, g1, be1, x4, wqkv_bf, bq2, wo_bf, bo2, sc2)

    y2d = y4.reshape(n, d)

    w1_bf = W1.astype(jnp.bfloat16)
    w2_bf = W2.astype(jnp.bfloat16)
    g2 = bn2_gamma.reshape(1, d)
    be2 = bn2_beta.reshape(1, d)
    b12 = b1.reshape(1, dff)
    b22 = b2.reshape(1, d)
    n_blk = 16
    blk = n // n_blk
    out2d = pl.pallas_call(
        functools.partial(_ffn_body, n_rows=n),
        grid=(n_blk,),
        in_specs=[
            pl.BlockSpec((bn * r // rpb, 2, d), lambda i: (0, 0, 0)),
            pl.BlockSpec((1, d), lambda i: (0, 0)),
            pl.BlockSpec((1, d), lambda i: (0, 0)),
            pl.BlockSpec((blk, d), lambda i: (i, 0)),
            pl.BlockSpec((d, dff), lambda i: (0, 0)),
            pl.BlockSpec((1, dff), lambda i: (0, 0)),
            pl.BlockSpec((dff, d), lambda i: (0, 0)),
            pl.BlockSpec((1, d), lambda i: (0, 0)),
            pl.BlockSpec((1, 1), lambda i: (0, 0)),
        ],
        out_specs=pl.BlockSpec((blk, d), lambda i: (i, 0)),
        out_shape=jax.ShapeDtypeStruct((n, d), jnp.float32),
    )(stats2, g2, be2, y2d, w1_bf, b12, w2_bf, b22, sc2)

    return out2d.reshape(Lt_s, b_s, L_s, d)


# speedup vs baseline: 1.0103x; 1.0064x over previous
import functools

import jax
import jax.numpy as jnp
from jax import lax
from jax.experimental import pallas as pl
from jax.experimental.pallas import tpu as pltpu
from jax.experimental.pallas import tpu_sc as plsc

HEADS = 12
NWIN = 16
TOPK = 4
TAU = 2.0
VTH = 1.0
EPS = 1e-5

def _stats_route_body(g_ref, be_ref, x_ref, wqk_ref, bqk_ref,
                      o_ref, a_ref, ms_scr, x_scr, *, n_blocks, n_rows,
                      w, d, bn, r):
    i = pl.program_id(0)
    rows_per_blk = n_rows // n_blocks
    regs_per_blk = rows_per_blk // w

    @pl.when(i < n_blocks)
    def _stats():
        xb = x_ref[...]
        part = jnp.stack(
            [jnp.sum(xb, axis=0), jnp.sum(xb * xb, axis=0)], axis=0)
        o_ref[pl.ds(i, 1)] = part[None]
        x_scr[pl.ds(i * rows_per_blk, rows_per_blk), :] = xb

    @pl.when(i >= n_blocks)
    def _ms():
        blk = i - n_blocks
        ssum = jnp.sum(o_ref[...], axis=0)
        mean = ssum[0] * (1.0 / n_rows)
        var = ssum[1] * (1.0 / n_rows) - mean * mean
        scl = g_ref[0] * jax.lax.rsqrt(var + EPS)
        shf = be_ref[0] - mean * scl
        xb = x_scr[pl.ds(blk * rows_per_blk, rows_per_blk), :]
        spk = _spike(xb * scl[None, :] + shf[None, :])
        for q in range(regs_per_blk):
            ms_scr[pl.ds(blk * regs_per_blk + q, 1), :] = (
                jnp.sum(spk[q * w:(q + 1) * w], axis=0)[None, :] * (1.0 / w))

        @pl.when(blk == n_blocks - 1)
        def _affinity():
            ms = ms_scr[...]
            qkr = jax.lax.dot_general(
                ms, wqk_ref[...], (((1,), (0,)), ((), ())),
                preferred_element_type=jnp.float32) + bqk_ref[...]
            qr = qkr[:, :d]
            kr = qkr[:, d:]
            for bb in range(bn):
                a_ref[bb] = jax.lax.dot_general(
                    qr[bb * r:(bb + 1) * r], kr[bb * r:(bb + 1) * r],
                    (((1,), (1,)), ((), ())),
                    preferred_element_type=jnp.float32)


_GATHER_DNUMS = lax.GatherDimensionNumbers(
    offset_dims=(), collapsed_slice_dims=(0,), start_index_map=(0,))


def _vgather(v, perm):
    return lax.gather(v, perm[:, None], _GATHER_DNUMS, (1,),
                      mode=lax.GatherScatterMode.PROMISE_IN_BOUNDS)


def _sc_topk(a_hbm, out_hbm, a_v, idx_v):
    nc = 2
    wid = lax.axis_index("s") * nc + lax.axis_index("c")
    ids = lax.iota(jnp.int32, 16)
    for t in range(2):
        row = wid * 2 + t
        pltpu.sync_copy(a_hbm.at[row], a_v)
        keys = a_v[...]
        out = ids * 0
        for p in range(TOPK):
            mx = keys
            for sft in (8, 4, 2, 1):
                mx = jnp.maximum(mx, _vgather(mx, jnp.bitwise_xor(ids, sft)))
            cand = jnp.where(keys >= mx, ids, 16)
            for sft in (8, 4, 2, 1):
                cand = jnp.minimum(
                    cand, _vgather(cand, jnp.bitwise_xor(ids, sft)))
            out = out + jnp.where(ids == p, cand, 0)
            keys = jnp.where(ids == cand, jnp.float32(-3.0e38), keys)
        idx_v[...] = out
        pltpu.sync_copy(idx_v, out_hbm.at[row])


def _bn_coeffs(stats_ref, g_ref, be_ref, n_rows):
    s = jnp.sum(stats_ref[...], axis=0)
    mean = s[0] * (1.0 / n_rows)
    var = s[1] * (1.0 / n_rows) - mean * mean
    scl = g_ref[0] * jax.lax.rsqrt(var + EPS)
    shf = be_ref[0] - mean * scl
    return scl, shf


def _spike(xn):
    v = xn / TAU
    return (v - VTH >= 0.0).astype(jnp.float32)


def _mega_body(idx_ref, stats_ref, g_ref, be_ref, xb_ref, wb_ref,
               bq_ref, wo_ref, bo_ref, sc_ref,
               y_ref, st_ref, q_scr, kv_scr,
               *, n_rows, w, d, r, dh, rpb):
    i = pl.program_id(0)
    nq = r // rpb
    c = jax.lax.rem(i, 2 * nq)
    bidx = i // (2 * nq)

    @pl.when(c < nq)
    def _qkv():
        scl, shf = _bn_coeffs(stats_ref, g_ref, be_ref, n_rows)
        xb = xb_ref[0].reshape(rpb * w, d)
        spk = _spike(xb * scl[None, :] + shf[None, :])
        qkv = jax.lax.dot_general(
            spk.astype(jnp.bfloat16), wb_ref[...],
            (((1,), (0,)), ((), ())), preferred_element_type=jnp.float32)
        qkv = qkv + bq_ref[...]
        q_scr[pl.ds(rpb * c, rpb)] = (
            qkv[:, :d].astype(jnp.bfloat16).reshape(rpb, w, d))
        kv_scr[pl.ds(rpb * c, rpb)] = (
            qkv[:, d:].astype(jnp.bfloat16).reshape(rpb, w, 2 * d))

    @pl.when(c >= nq)
    def _attn():
        ja = c - nq
        qs = []
        kcs = []
        vcs = []
        for u in range(rpb):
            reg = rpb * ja + u
            qs.append(q_scr[reg] * jnp.bfloat16(dh ** -0.5))
            kvc = jnp.concatenate(
                [kv_scr[idx_ref[bidx, reg, t]] for t in range(TOPK)],
                axis=0)
            kcs.append(kvc[:, :d])
            vcs.append(kvc[:, d:])
        nk = kcs[0].shape[0]
        ones_m = jnp.ones((nk, 8), jnp.bfloat16)
        heads = [[] for _ in range(rpb)]
        for h in range(HEADS):
            sl = slice(h * dh, (h + 1) * dh)
            for u in range(rpb):
                s = jax.lax.dot_general(
                    qs[u][:, sl], kcs[u][:, sl], (((1,), (1,)), ((), ())),
                    preferred_element_type=jnp.float32).astype(jnp.bfloat16)
                m = jnp.max(s, axis=1, keepdims=True)
                p = jnp.exp(s - m)
                l = jax.lax.dot_general(
                    p, ones_m, (((1,), (0,)), ((), ())),
                    preferred_element_type=jnp.float32)[:, :1]
                o_h = jax.lax.dot_general(
                    p, vcs[u][:, sl], (((1,), (0,)), ((), ())),
                    preferred_element_type=jnp.float32)
                heads[u].append((o_h * (1.0 / l)).astype(jnp.bfloat16))
        ssum = None
        ssq = None
        for u in range(rpb):
            oc = jnp.concatenate(heads[u], axis=1)
            res = jax.lax.dot_general(
                oc, wo_ref[...], (((1,), (0,)), ((), ())),
                preferred_element_type=jnp.float32) + bo_ref[...]
            y = xb_ref[0, u] + sc_ref[0, 0] * res
            y_ref[0, u] = y
            us = jnp.sum(y, axis=0)
            uq = jnp.sum(y * y, axis=0)
            ssum = us if ssum is None else ssum + us
            ssq = uq if ssq is None else ssq + uq
        st_ref[0, 0, :] = ssum
        st_ref[0, 1, :] = ssq


def _ffn_body(stats_ref, g_ref, be_ref, y_ref, w1_ref, b1_ref,
              w2_ref, b2_ref, sc_ref, o_ref, *, n_rows):
    scl, shf = _bn_coeffs(stats_ref, g_ref, be_ref, n_rows)
    yb = y_ref[...]
    spk = _spike(yb * scl[None, :] + shf[None, :])
    h = jax.lax.dot_general(
        spk.astype(jnp.bfloat16), w1_ref[...], (((1,), (0,)), ((), ())),
        preferred_element_type=jnp.float32) + b1_ref[...]
    g = 0.5 * h * (1.0 + jax.lax.erf(h * (2.0 ** -0.5)))
    f = jax.lax.dot_general(
        g.astype(jnp.bfloat16), w2_ref[...], (((1,), (0,)), ((), ())),
        preferred_element_type=jnp.float32) + b2_ref[...]
    o_ref[...] = yb + sc_ref[0, 0] * f


def kernel(x, Lt, b, L, dim, bn1_gamma, bn1_beta, W_qkv, b_qkv, W_o, b_o,
           bn2_gamma, bn2_beta, W1, b1, W2, b2, scale):
    Lt_s, b_s, L_s, d = x.shape
    bn = Lt_s * b_s
    r = NWIN
    w = L_s // r
    n = bn * L_s
    dh = d // HEADS
    dff = W1.shape[1]
    rpb = 4
    cyc = 2 * (r // rpb)

    x2d = x.reshape(n, d)
    x4 = x.reshape(bn, r, w, d)
    n_blocks = 8

    g1 = bn1_gamma.reshape(1, d)
    be1 = bn1_beta.reshape(1, d)
    wqk = W_qkv[:, :2 * d]
    bqk = b_qkv[:2 * d].reshape(1, 2 * d)

    stats1, a_mat = pl.pallas_call(
        functools.partial(_stats_route_body, n_blocks=n_blocks, n_rows=n,
                          w=w, d=d, bn=bn, r=r),
        grid=(2 * n_blocks,),
        in_specs=[
            pl.BlockSpec((1, d), lambda i: (0, 0)),
            pl.BlockSpec((1, d), lambda i: (0, 0)),
            pl.BlockSpec((n // n_blocks, d),
                         lambda i: (jnp.minimum(i, n_blocks - 1), 0)),
            pl.BlockSpec((d, 2 * d), lambda i: (0, 0)),
            pl.BlockSpec((1, 2 * d), lambda i: (0, 0)),
        ],
        out_specs=[
            pl.BlockSpec((n_blocks, 2, d), lambda i: (0, 0, 0)),
            pl.BlockSpec((bn, r, r), lambda i: (0, 0, 0)),
        ],
        out_shape=[
            jax.ShapeDtypeStruct((n_blocks, 2, d), jnp.float32),
            jax.ShapeDtypeStruct((bn, r, r), jnp.float32),
        ],
        scratch_shapes=[pltpu.VMEM((bn * r, d), jnp.float32),
                        pltpu.VMEM((n, d), jnp.float32)],
    )(g1, be1, x2d, wqk, bqk)

    idxfull = pl.kernel(
        _sc_topk,
        out_type=jax.ShapeDtypeStruct((bn * r, 16), jnp.int32),
        mesh=plsc.VectorSubcoreMesh(core_axis_name="c",
                                    subcore_axis_name="s"),
        scratch_types=[pltpu.VMEM((16,), jnp.float32),
                       pltpu.VMEM((16,), jnp.int32)],
    )(a_mat.reshape(bn * r, r))
    idx = idxfull.reshape(bn, r, 16)[:, :, :TOPK]

    wqkv_bf = W_qkv.astype(jnp.bfloat16)
    bq2 = b_qkv.reshape(1, 3 * d)
    wo_bf = W_o.astype(jnp.bfloat16)
    bo2 = b_o.reshape(1, d)
    sc2 = scale.reshape(1, 1)

    nq = r // rpb

    def _b(i):
        return i // cyc

    def _c(i):
        return jax.lax.rem(i, cyc)

    def x_map(i, s):
        return (_b(i), jax.lax.rem(_c(i), nq), 0, 0)

    def att_map(i, s):
        return (_b(i), jnp.clip(_c(i) - nq, 0, nq - 1), 0, 0)

    def st_map(i, s):
        return (_b(i) * nq + jnp.clip(_c(i) - nq, 0, nq - 1), 0, 0)

    const2 = lambda i, s: (0, 0)
    const3 = lambda i, s: (0, 0, 0)
    y4, stats2 = pl.pallas_call(
        functools.partial(_mega_body, n_rows=n, w=w, d=d, r=r, dh=dh,
                          rpb=rpb),
        grid_spec=pltpu.PrefetchScalarGridSpec(
            num_scalar_prefetch=1,
            grid=(bn * cyc,),
            in_specs=[
                pl.BlockSpec((n_blocks, 2, d), const3),
                pl.BlockSpec((1, d), const2),
                pl.BlockSpec((1, d), const2),
                pl.BlockSpec((1, rpb, w, d), x_map),
                pl.BlockSpec((d, 3 * d), const2),
                pl.BlockSpec((1, 3 * d), const2),
                pl.BlockSpec((d, d), const2),
                pl.BlockSpec((1, d), const2),
                pl.BlockSpec((1, 1), const2),
            ],
            out_specs=[
                pl.BlockSpec((1, rpb, w, d), att_map),
                pl.BlockSpec((1, 2, d), st_map),
            ],
            scratch_shapes=[
                pltpu.VMEM((r, w, d), jnp.bfloat16),
                pltpu.VMEM((r, w, 2 * d), jnp.bfloat16),
            ],
        ),
        out_shape=[
            jax.ShapeDtypeStruct((bn, r, w, d), jnp.float32),
            jax.ShapeDtypeStruct((bn * r // rpb, 2, d), jnp.float32),
        ],
    )(idx, stats1, g1, be1, x4, wqkv_bf, bq2, wo_bf, bo2, sc2)

    y2d = y4.reshape(n, d)

    w1_bf = W1.astype(jnp.bfloat16)
    w2_bf = W2.astype(jnp.bfloat16)
    g2 = bn2_gamma.reshape(1, d)
    be2 = bn2_beta.reshape(1, d)
    b12 = b1.reshape(1, dff)
    b22 = b2.reshape(1, d)
    n_blk = 8
    blk = n // n_blk
    out2d = pl.pallas_call(
        functools.partial(_ffn_body, n_rows=n),
        grid=(n_blk,),
        in_specs=[
            pl.BlockSpec((bn * r // rpb, 2, d), lambda i: (0, 0, 0)),
            pl.BlockSpec((1, d), lambda i: (0, 0)),
            pl.BlockSpec((1, d), lambda i: (0, 0)),
            pl.BlockSpec((blk, d), lambda i: (i, 0)),
            pl.BlockSpec((d, dff), lambda i: (0, 0)),
            pl.BlockSpec((1, dff), lambda i: (0, 0)),
            pl.BlockSpec((dff, d), lambda i: (0, 0)),
            pl.BlockSpec((1, d), lambda i: (0, 0)),
            pl.BlockSpec((1, 1), lambda i: (0, 0)),
        ],
        out_specs=pl.BlockSpec((blk, d), lambda i: (i, 0)),
        out_shape=jax.ShapeDtypeStruct((n, d), jnp.float32),
    )(stats2, g2, be2, y2d, w1_bf, b12, w2_bf, b22, sc2)

    return out2d.reshape(Lt_s, b_s, L_s, d)


# speedup vs baseline: 1.0137x; 1.0034x over previous
import functools

import jax
import jax.numpy as jnp
from jax import lax
from jax.experimental import pallas as pl
from jax.experimental.pallas import tpu as pltpu
from jax.experimental.pallas import tpu_sc as plsc

HEADS = 12
NWIN = 16
TOPK = 4
TAU = 2.0
VTH = 1.0
EPS = 1e-5

def _stats_route_body(g_ref, be_ref, x_ref, wqk_ref, bqk_ref,
                      o_ref, a_ref, ms_scr, x_scr, *, n_blocks, n_rows,
                      w, d, bn, r):
    i = pl.program_id(0)
    rows_per_blk = n_rows // n_blocks
    regs_per_blk = rows_per_blk // w

    @pl.when(i < n_blocks)
    def _stats():
        xb = x_ref[...]
        part = jnp.stack(
            [jnp.sum(xb, axis=0), jnp.sum(xb * xb, axis=0)], axis=0)
        o_ref[pl.ds(i, 1)] = part[None]
        x_scr[pl.ds(i * rows_per_blk, rows_per_blk), :] = xb

    @pl.when(i >= n_blocks)
    def _ms():
        blk = i - n_blocks
        ssum = jnp.sum(o_ref[...], axis=0)
        mean = ssum[0] * (1.0 / n_rows)
        var = ssum[1] * (1.0 / n_rows) - mean * mean
        scl = g_ref[0] * jax.lax.rsqrt(var + EPS)
        shf = be_ref[0] - mean * scl
        xb = x_scr[pl.ds(blk * rows_per_blk, rows_per_blk), :]
        spk = _spike(xb * scl[None, :] + shf[None, :])
        for q in range(regs_per_blk):
            ms_scr[pl.ds(blk * regs_per_blk + q, 1), :] = (
                jnp.sum(spk[q * w:(q + 1) * w], axis=0)[None, :] * (1.0 / w))

        @pl.when(blk == n_blocks - 1)
        def _affinity():
            ms = ms_scr[...]
            qkr = jax.lax.dot_general(
                ms, wqk_ref[...], (((1,), (0,)), ((), ())),
                preferred_element_type=jnp.float32) + bqk_ref[...]
            qr = qkr[:, :d]
            kr = qkr[:, d:]
            for bb in range(bn):
                a_ref[bb] = jax.lax.dot_general(
                    qr[bb * r:(bb + 1) * r], kr[bb * r:(bb + 1) * r],
                    (((1,), (1,)), ((), ())),
                    preferred_element_type=jnp.float32)


_GATHER_DNUMS = lax.GatherDimensionNumbers(
    offset_dims=(), collapsed_slice_dims=(0,), start_index_map=(0,))


def _vgather(v, perm):
    return lax.gather(v, perm[:, None], _GATHER_DNUMS, (1,),
                      mode=lax.GatherScatterMode.PROMISE_IN_BOUNDS)


def _sc_topk(a_hbm, out_hbm, a_v, idx_v):
    nc = 2
    wid = lax.axis_index("s") * nc + lax.axis_index("c")
    ids = lax.iota(jnp.int32, 16)
    for t in range(2):
        row = wid * 2 + t
        pltpu.sync_copy(a_hbm.at[row], a_v)
        keys = a_v[...]
        out = ids * 0
        for p in range(TOPK):
            mx = keys
            for sft in (8, 4, 2, 1):
                mx = jnp.maximum(mx, _vgather(mx, jnp.bitwise_xor(ids, sft)))
            cand = jnp.where(keys >= mx, ids, 16)
            for sft in (8, 4, 2, 1):
                cand = jnp.minimum(
                    cand, _vgather(cand, jnp.bitwise_xor(ids, sft)))
            out = out + jnp.where(ids == p, cand, 0)
            keys = jnp.where(ids == cand, jnp.float32(-3.0e38), keys)
        idx_v[...] = out
        pltpu.sync_copy(idx_v, out_hbm.at[row])


def _bn_coeffs(stats_ref, g_ref, be_ref, n_rows):
    s = jnp.sum(stats_ref[...], axis=0)
    mean = s[0] * (1.0 / n_rows)
    var = s[1] * (1.0 / n_rows) - mean * mean
    scl = g_ref[0] * jax.lax.rsqrt(var + EPS)
    shf = be_ref[0] - mean * scl
    return scl, shf


def _spike(xn):
    v = xn / TAU
    return (v - VTH >= 0.0).astype(jnp.float32)


def _mega_body(idx_ref, stats_ref, g_ref, be_ref, xb_ref, wb_ref,
               bq_ref, wo_ref, bo_ref, sc_ref,
               y_ref, st_ref, q_scr, kv_scr,
               *, n_rows, w, d, r, dh, rpb):
    i = pl.program_id(0)
    nq = r // rpb
    c = jax.lax.rem(i, 2 * nq)
    bidx = i // (2 * nq)

    @pl.when(c < nq)
    def _qkv():
        scl, shf = _bn_coeffs(stats_ref, g_ref, be_ref, n_rows)
        xb = xb_ref[0].reshape(rpb * w, d)
        spk = _spike(xb * scl[None, :] + shf[None, :])
        qkv = jax.lax.dot_general(
            spk.astype(jnp.bfloat16), wb_ref[...],
            (((1,), (0,)), ((), ())), preferred_element_type=jnp.float32)
        qkv = qkv + bq_ref[...]
        q_scr[pl.ds(rpb * c, rpb)] = (
            qkv[:, :d].astype(jnp.bfloat16).reshape(rpb, w, d))
        kv_scr[pl.ds(rpb * c, rpb)] = (
            qkv[:, d:].astype(jnp.bfloat16).reshape(rpb, w, 2 * d))

    @pl.when(c >= nq)
    def _attn():
        ja = c - nq
        qs = []
        kcs = []
        vcs = []
        for u in range(rpb):
            reg = rpb * ja + u
            qs.append(q_scr[reg] * jnp.bfloat16(dh ** -0.5))
            kvc = jnp.concatenate(
                [kv_scr[idx_ref[bidx, reg, t]] for t in range(TOPK)],
                axis=0)
            kcs.append(kvc[:, :d])
            vcs.append(kvc[:, d:])
        nk = kcs[0].shape[0]
        ones_m = jnp.ones((nk, 8), jnp.bfloat16)
        heads = [[] for _ in range(rpb)]
        for h in range(HEADS):
            sl = slice(h * dh, (h + 1) * dh)
            for u in range(rpb):
                s = jax.lax.dot_general(
                    qs[u][:, sl], kcs[u][:, sl], (((1,), (1,)), ((), ())),
                    preferred_element_type=jnp.float32).astype(jnp.bfloat16)
                m = jnp.max(s, axis=1, keepdims=True)
                p = jnp.exp(s - m)
                l = jax.lax.dot_general(
                    p, ones_m, (((1,), (0,)), ((), ())),
                    preferred_element_type=jnp.float32)[:, :1]
                o_h = jax.lax.dot_general(
                    p, vcs[u][:, sl], (((1,), (0,)), ((), ())),
                    preferred_element_type=jnp.float32)
                heads[u].append((o_h * (1.0 / l)).astype(jnp.bfloat16))
        ssum = None
        ssq = None
        for u in range(rpb):
            oc = jnp.concatenate(heads[u], axis=1)
            res = jax.lax.dot_general(
                oc, wo_ref[...], (((1,), (0,)), ((), ())),
                preferred_element_type=jnp.float32) + bo_ref[...]
            y = xb_ref[0, u] + sc_ref[0, 0] * res
            y_ref[0, u] = y
            us = jnp.sum(y, axis=0)
            uq = jnp.sum(y * y, axis=0)
            ssum = us if ssum is None else ssum + us
            ssq = uq if ssq is None else ssq + uq
        st_ref[0, 0, :] = ssum
        st_ref[0, 1, :] = ssq


def _ffn_body(stats_ref, g_ref, be_ref, y_ref, w1_ref, b1_ref,
              w2_ref, b2_ref, sc_ref, o_ref, *, n_rows):
    scl, shf = _bn_coeffs(stats_ref, g_ref, be_ref, n_rows)
    yb = y_ref[...]
    spk = _spike(yb * scl[None, :] + shf[None, :])
    h = jax.lax.dot_general(
        spk.astype(jnp.bfloat16), w1_ref[...], (((1,), (0,)), ((), ())),
        preferred_element_type=jnp.float32) + b1_ref[...]
    g = 0.5 * h * (1.0 + jax.lax.erf(h * (2.0 ** -0.5)))
    f = jax.lax.dot_general(
        g.astype(jnp.bfloat16), w2_ref[...], (((1,), (0,)), ((), ())),
        preferred_element_type=jnp.float32) + b2_ref[...]
    o_ref[...] = yb + sc_ref[0, 0] * f


def kernel(x, Lt, b, L, dim, bn1_gamma, bn1_beta, W_qkv, b_qkv, W_o, b_o,
           bn2_gamma, bn2_beta, W1, b1, W2, b2, scale):
    Lt_s, b_s, L_s, d = x.shape
    bn = Lt_s * b_s
    r = NWIN
    w = L_s // r
    n = bn * L_s
    dh = d // HEADS
    dff = W1.shape[1]
    rpb = 4
    cyc = 2 * (r // rpb)

    x2d = x.reshape(n, d)
    x4 = x.reshape(bn, r, w, d)
    n_blocks = 4

    g1 = bn1_gamma.reshape(1, d)
    be1 = bn1_beta.reshape(1, d)
    wqk = W_qkv[:, :2 * d]
    bqk = b_qkv[:2 * d].reshape(1, 2 * d)

    stats1, a_mat = pl.pallas_call(
        functools.partial(_stats_route_body, n_blocks=n_blocks, n_rows=n,
                          w=w, d=d, bn=bn, r=r),
        grid=(2 * n_blocks,),
        in_specs=[
            pl.BlockSpec((1, d), lambda i: (0, 0)),
            pl.BlockSpec((1, d), lambda i: (0, 0)),
            pl.BlockSpec((n // n_blocks, d),
                         lambda i: (jnp.minimum(i, n_blocks - 1), 0)),
            pl.BlockSpec((d, 2 * d), lambda i: (0, 0)),
            pl.BlockSpec((1, 2 * d), lambda i: (0, 0)),
        ],
        out_specs=[
            pl.BlockSpec((n_blocks, 2, d), lambda i: (0, 0, 0)),
            pl.BlockSpec((bn, r, r), lambda i: (0, 0, 0)),
        ],
        out_shape=[
            jax.ShapeDtypeStruct((n_blocks, 2, d), jnp.float32),
            jax.ShapeDtypeStruct((bn, r, r), jnp.float32),
        ],
        scratch_shapes=[pltpu.VMEM((bn * r, d), jnp.float32),
                        pltpu.VMEM((n, d), jnp.float32)],
    )(g1, be1, x2d, wqk, bqk)

    idxfull = pl.kernel(
        _sc_topk,
        out_type=jax.ShapeDtypeStruct((bn * r, 16), jnp.int32),
        mesh=plsc.VectorSubcoreMesh(core_axis_name="c",
                                    subcore_axis_name="s"),
        scratch_types=[pltpu.VMEM((16,), jnp.float32),
                       pltpu.VMEM((16,), jnp.int32)],
    )(a_mat.reshape(bn * r, r))
    idx = idxfull.reshape(bn, r, 16)[:, :, :TOPK]

    wqkv_bf = W_qkv.astype(jnp.bfloat16)
    bq2 = b_qkv.reshape(1, 3 * d)
    wo_bf = W_o.astype(jnp.bfloat16)
    bo2 = b_o.reshape(1, d)
    sc2 = scale.reshape(1, 1)

    nq = r // rpb

    def _b(i):
        return i // cyc

    def _c(i):
        return jax.lax.rem(i, cyc)

    def x_map(i, s):
        return (_b(i), jax.lax.rem(_c(i), nq), 0, 0)

    def att_map(i, s):
        return (_b(i), jnp.clip(_c(i) - nq, 0, nq - 1), 0, 0)

    def st_map(i, s):
        return (_b(i) * nq + jnp.clip(_c(i) - nq, 0, nq - 1), 0, 0)

    const2 = lambda i, s: (0, 0)
    const3 = lambda i, s: (0, 0, 0)
    y4, stats2 = pl.pallas_call(
        functools.partial(_mega_body, n_rows=n, w=w, d=d, r=r, dh=dh,
                          rpb=rpb),
        grid_spec=pltpu.PrefetchScalarGridSpec(
            num_scalar_prefetch=1,
            grid=(bn * cyc,),
            in_specs=[
                pl.BlockSpec((n_blocks, 2, d), const3),
                pl.BlockSpec((1, d), const2),
                pl.BlockSpec((1, d), const2),
                pl.BlockSpec((1, rpb, w, d), x_map),
                pl.BlockSpec((d, 3 * d), const2),
                pl.BlockSpec((1, 3 * d), const2),
                pl.BlockSpec((d, d), const2),
                pl.BlockSpec((1, d), const2),
                pl.BlockSpec((1, 1), const2),
            ],
            out_specs=[
                pl.BlockSpec((1, rpb, w, d), att_map),
                pl.BlockSpec((1, 2, d), st_map),
            ],
            scratch_shapes=[
                pltpu.VMEM((r, w, d), jnp.bfloat16),
                pltpu.VMEM((r, w, 2 * d), jnp.bfloat16),
            ],
        ),
        out_shape=[
            jax.ShapeDtypeStruct((bn, r, w, d), jnp.float32),
            jax.ShapeDtypeStruct((bn * r // rpb, 2, d), jnp.float32),
        ],
    )(idx, stats1, g1, be1, x4, wqkv_bf, bq2, wo_bf, bo2, sc2)

    y2d = y4.reshape(n, d)

    w1_bf = W1.astype(jnp.bfloat16)
    w2_bf = W2.astype(jnp.bfloat16)
    g2 = bn2_gamma.reshape(1, d)
    be2 = bn2_beta.reshape(1, d)
    b12 = b1.reshape(1, dff)
    b22 = b2.reshape(1, d)
    n_blk = 8
    blk = n // n_blk
    out2d = pl.pallas_call(
        functools.partial(_ffn_body, n_rows=n),
        grid=(n_blk,),
        in_specs=[
            pl.BlockSpec((bn * r // rpb, 2, d), lambda i: (0, 0, 0)),
            pl.BlockSpec((1, d), lambda i: (0, 0)),
            pl.BlockSpec((1, d), lambda i: (0, 0)),
            pl.BlockSpec((blk, d), lambda i: (i, 0)),
            pl.BlockSpec((d, dff), lambda i: (0, 0)),
            pl.BlockSpec((1, dff), lambda i: (0, 0)),
            pl.BlockSpec((dff, d), lambda i: (0, 0)),
            pl.BlockSpec((1, d), lambda i: (0, 0)),
            pl.BlockSpec((1, 1), lambda i: (0, 0)),
        ],
        out_specs=pl.BlockSpec((blk, d), lambda i: (i, 0)),
        out_shape=jax.ShapeDtypeStruct((n, d), jnp.float32),
    )(stats2, g2, be2, y2d, w1_bf, b12, w2_bf, b22, sc2)

    return out2d.reshape(Lt_s, b_s, L_s, d)
